# baseline (speedup 1.0000x reference)
import functools

import jax
import jax.numpy as jnp
from jax import lax
from jax.experimental import pallas as pl
from jax.experimental.pallas import tpu as pltpu
from jax.experimental.pallas import tpu_sc as plsc

N_NODES = 10000
N_EDGES = 320000
HEADS = 8
DK = 16
D = 128

NC = 2
NS = 16
NW = NC * NS
EPT = N_EDGES // NW
CH = 80
NCHUNK = EPT // CH
NPAD = 10240
NPT = NPAD // NS

_BN = 2000
_BE = 2560

_f32 = jnp.float32


def _ln(x, g, b):
    m = jnp.mean(x, axis=1, keepdims=True)
    v = jnp.mean((x - m) * (x - m), axis=1, keepdims=True)
    return (x - m) * lax.rsqrt(v + 1e-5) * g + b


def _dot(a, b):
    return jnp.dot(a, b, preferred_element_type=_f32)


def _bdot(a, b):
    return jnp.dot(a.astype(jnp.bfloat16), b.astype(jnp.bfloat16),
                   preferred_element_type=_f32)


def _embed_h_body(atom_ref, lap_ref, wh, bh, wl, bl, out_ref):
    out_ref[...] = (_dot(atom_ref[...], wh[...]) + bh[...]
                    + _dot(lap_ref[...], wl[...]) + bl[...])


def _embed_h(atom, lap, wh, bh, wl, bl):
    grid = (N_NODES // _BN,)
    return pl.pallas_call(
        _embed_h_body,
        grid=grid,
        in_specs=[
            pl.BlockSpec((_BN, 128), lambda i: (i, 0)),
            pl.BlockSpec((_BN, 16), lambda i: (i, 0)),
            pl.BlockSpec((128, 128), lambda i: (0, 0)),
            pl.BlockSpec((1, 128), lambda i: (0, 0)),
            pl.BlockSpec((16, 128), lambda i: (0, 0)),
            pl.BlockSpec((1, 128), lambda i: (0, 0)),
        ],
        out_specs=pl.BlockSpec((_BN, 128), lambda i: (i, 0)),
        out_shape=jax.ShapeDtypeStruct((N_NODES, 128), _f32),
    )(atom, lap, wh, bh, wl, bl)


def _embed_e_body(bond_ref, we, be, out_ref):
    out_ref[...] = _dot(bond_ref[...], we[...]) + be[...]


def _embed_e(bond, we, be):
    grid = (N_EDGES // _BE,)
    return pl.pallas_call(
        _embed_e_body,
        grid=grid,
        in_specs=[
            pl.BlockSpec((_BE, 16), lambda i: (i, 0)),
            pl.BlockSpec((16, 128), lambda i: (0, 0)),
            pl.BlockSpec((1, 128), lambda i: (0, 0)),
        ],
        out_specs=pl.BlockSpec((_BE, 128), lambda i: (i, 0)),
        out_shape=jax.ShapeDtypeStruct((N_EDGES, 128), _f32),
    )(bond, we, be)


def _proj_body(h_ref, wq, bq, wk, bk, wv, bv, q_ref, k_ref, v_ref):
    h = h_ref[...]
    q_ref[...] = (_dot(h, wq[...]) + bq[...]) * 0.25
    k_ref[...] = _dot(h, wk[...]) + bk[...]
    v_ref[...] = _dot(h, wv[...]) + bv[...]


def _proj(h, wq, bq, wk, bk, wv, bv):
    grid = (N_NODES // _BN,)
    wspec = pl.BlockSpec((128, 128), lambda i: (0, 0))
    bspec = pl.BlockSpec((1, 128), lambda i: (0, 0))
    ospec = pl.BlockSpec((_BN, 128), lambda i: (i, 0))
    oshape = jax.ShapeDtypeStruct((N_NODES, 128), _f32)
    return pl.pallas_call(
        _proj_body,
        grid=grid,
        in_specs=[pl.BlockSpec((_BN, 128), lambda i: (i, 0)),
                  wspec, bspec, wspec, bspec, wspec, bspec],
        out_specs=[ospec, ospec, ospec],
        out_shape=[oshape, oshape, oshape],
    )(h, wq, bq, wk, bk, wv, bv)


def _edge_body(e_ref, p_ref, vs_ref, we, be, woe, boe, w1, c1, w2, c2,
               g1, d1, g2, d2, e2_ref, av_ref, a_ref, a128_ref):
    e = e_ref[...]
    ee = _dot(e, we[...]) + be[...]
    score = p_ref[...] * ee
    parts = []
    for h in range(HEADS):
        parts.append(jnp.sum(score[:, h * DK:(h + 1) * DK], axis=1,
                             keepdims=True))
    s = jnp.concatenate(parts, axis=1)
    a = jnp.exp(jnp.clip(s, -5.0, 5.0))
    vs = vs_ref[...]
    av_parts = []
    for h in range(HEADS):
        av_parts.append(vs[:, h * DK:(h + 1) * DK] * a[:, h:h + 1])
    av_ref[...] = jnp.concatenate(av_parts, axis=1)
    a_ref[...] = jnp.concatenate([a, jnp.zeros_like(a)], axis=1)
    a128_ref[...] = jnp.concatenate(
        [a, jnp.zeros((a.shape[0], 120), _f32)], axis=1)
    e1 = _ln(e + _bdot(score, woe[...]) + boe[...], g1[...], d1[...])
    ffn = _bdot(jnp.maximum(_bdot(e1, w1[...]) + c1[...], 0.0), w2[...]) + c2[...]
    e2_ref[...] = _ln(e1 + ffn, g2[...], d2[...])


def _edge(e, p, vs, we, be, woe, boe, w1, c1, w2, c2, g1, d1, g2, d2):
    grid = (N_EDGES // _BE,)
    espec = pl.BlockSpec((_BE, 128), lambda i: (i, 0))
    return pl.pallas_call(
        _edge_body,
        grid=grid,
        in_specs=[espec, espec, espec,
                  pl.BlockSpec((128, 128), lambda i: (0, 0)),
                  pl.BlockSpec((1, 128), lambda i: (0, 0)),
                  pl.BlockSpec((128, 128), lambda i: (0, 0)),
                  pl.BlockSpec((1, 128), lambda i: (0, 0)),
                  pl.BlockSpec((128, 256), lambda i: (0, 0)),
                  pl.BlockSpec((1, 256), lambda i: (0, 0)),
                  pl.BlockSpec((256, 128), lambda i: (0, 0)),
                  pl.BlockSpec((1, 128), lambda i: (0, 0)),
                  pl.BlockSpec((1, 128), lambda i: (0, 0)),
                  pl.BlockSpec((1, 128), lambda i: (0, 0)),
                  pl.BlockSpec((1, 128), lambda i: (0, 0)),
                  pl.BlockSpec((1, 128), lambda i: (0, 0))],
        out_specs=[espec, espec, pl.BlockSpec((_BE, 16), lambda i: (i, 0)),
                   espec],
        out_shape=[jax.ShapeDtypeStruct((N_EDGES, 128), _f32),
                   jax.ShapeDtypeStruct((N_EDGES, 128), _f32),
                   jax.ShapeDtypeStruct((N_EDGES, 16), _f32),
                   jax.ShapeDtypeStruct((N_EDGES, 128), _f32)],
    )(e, p, vs, we, be, woe, boe, w1, c1, w2, c2, g1, d1, g2, d2)


def _node_body(wvp_ref, zp_ref, h_ref, wo, bo, w1, c1, w2, c2,
               g1, d1, g2, d2, h2_ref, zr_ref):
    wv = wvp_ref[0] + wvp_ref[1]
    z = (zp_ref[0] + zp_ref[1])[:, :16]
    zr = 1.0 / (z + 1e-6)
    zr_ref[...] = jnp.concatenate(
        [zr, jnp.zeros((zr.shape[0], 112), _f32)], axis=1)
    parts = []
    for h in range(HEADS):
        parts.append(wv[:, h * DK:(h + 1) * DK] * zr[:, h:h + 1])
    h_attn = jnp.concatenate(parts, axis=1)
    hh = h_ref[...]
    h1 = _ln(hh + _dot(h_attn, wo[...]) + bo[...], g1[...], d1[...])
    ffn = _dot(jnp.maximum(_dot(h1, w1[...]) + c1[...], 0.0), w2[...]) + c2[...]
    h2_ref[...] = _ln(h1 + ffn, g2[...], d2[...])


def _node(wvp, zp, h, wo, bo, w1, c1, w2, c2, g1, d1, g2, d2):
    grid = (N_NODES // _BN,)
    return pl.pallas_call(
        _node_body,
        grid=grid,
        in_specs=[pl.BlockSpec((2, _BN, 128), lambda i: (0, i, 0)),
                  pl.BlockSpec((2, _BN, 128), lambda i: (0, i, 0)),
                  pl.BlockSpec((_BN, 128), lambda i: (i, 0)),
                  pl.BlockSpec((128, 128), lambda i: (0, 0)),
                  pl.BlockSpec((1, 128), lambda i: (0, 0)),
                  pl.BlockSpec((128, 256), lambda i: (0, 0)),
                  pl.BlockSpec((1, 256), lambda i: (0, 0)),
                  pl.BlockSpec((256, 128), lambda i: (0, 0)),
                  pl.BlockSpec((1, 128), lambda i: (0, 0)),
                  pl.BlockSpec((1, 128), lambda i: (0, 0)),
                  pl.BlockSpec((1, 128), lambda i: (0, 0)),
                  pl.BlockSpec((1, 128), lambda i: (0, 0)),
                  pl.BlockSpec((1, 128), lambda i: (0, 0))],
        out_specs=[pl.BlockSpec((_BN, 128), lambda i: (i, 0)),
                   pl.BlockSpec((_BN, 128), lambda i: (i, 0))],
        out_shape=[jax.ShapeDtypeStruct((N_NODES, 128), _f32),
                   jax.ShapeDtypeStruct((NPAD, 128), _f32)],
    )(wvp, zp, h, wo, bo, w1, c1, w2, c2, g1, d1, g2, d2)


def _sc_mesh():
    return plsc.VectorSubcoreMesh(core_axis_name="c", subcore_axis_name="s")


def _gather_stage(k, q, v, src, dst):
    @functools.partial(
        pl.kernel,
        mesh=_sc_mesh(),
        out_type=[jax.ShapeDtypeStruct((N_EDGES, 128), _f32),
                  jax.ShapeDtypeStruct((N_EDGES, 128), _f32)],
        scratch_types=[pltpu.VMEM((CH,), jnp.int32),
                       pltpu.VMEM((CH,), jnp.int32),
                       pltpu.VMEM((CH, 128), _f32),
                       pltpu.VMEM((CH, 128), _f32),
                       pltpu.VMEM((CH, 128), _f32),
                       pltpu.VMEM((CH,), jnp.int32),
                       pltpu.VMEM((CH,), jnp.int32),
                       pltpu.VMEM((CH, 128), _f32),
                       pltpu.VMEM((CH, 128), _f32),
                       pltpu.VMEM((CH, 128), _f32),
                       pltpu.VMEM((CH,), jnp.int32),
                       pltpu.VMEM((CH,), jnp.int32),
                       pltpu.VMEM((CH, 128), _f32),
                       pltpu.VMEM((CH, 128), _f32),
                       pltpu.VMEM((CH, 128), _f32),
                       pltpu.SemaphoreType.DMA,
                       pltpu.SemaphoreType.DMA,
                       pltpu.SemaphoreType.DMA,
                       pltpu.SemaphoreType.DMA],
    )
    def body(k_hbm, q_hbm, v_hbm, src_hbm, dst_hbm, p_out, vs_out,
             siA, diA, kbA, qbA, vbA, siB, diB, kbB, qbB, vbB,
             siC, diC, kbC, qbC, vbC, gsA, gsB, gsC, osem):
        wid = lax.axis_index("s") * NC + lax.axis_index("c")
        base0 = wid * EPT

        def issue(base, si, di, kb, qb, vb, gsem):
            pltpu.sync_copy(src_hbm.at[pl.ds(base, CH)], si)
            pltpu.sync_copy(dst_hbm.at[pl.ds(base, CH)], di)
            return (pltpu.async_copy(k_hbm.at[si], kb, gsem),
                    pltpu.async_copy(q_hbm.at[di], qb, gsem),
                    pltpu.async_copy(v_hbm.at[si], vb, gsem))

        def mul(kb, qb):
            def row(r, carry2):
                for j in range(8):
                    sl = pl.ds(j * 16, 16)
                    kb[r, sl] = kb[r, sl] * qb[r, sl]
                return carry2

            lax.fori_loop(0, CH, row, 0)

        def drain_outs():
            for _ in range(6):
                pltpu.make_async_copy(
                    kbA, p_out.at[pl.ds(base0, CH)], osem).wait()

        def triple(i, carry):
            b0 = base0 + (3 * i) * CH

            @pl.when(i > 0)
            def _():
                drain_outs()

            hA = issue(b0, siA, diA, kbA, qbA, vbA, gsA)
            hB = issue(b0 + CH, siB, diB, kbB, qbB, vbB, gsB)
            hC = issue(b0 + 2 * CH, siC, diC, kbC, qbC, vbC, gsC)
            for h, kb, qb, vb, bo in ((hA, kbA, qbA, vbA, b0),
                                      (hB, kbB, qbB, vbB, b0 + CH),
                                      (hC, kbC, qbC, vbC, b0 + 2 * CH)):
                for x in h:
                    x.wait()
                mul(kb, qb)
                pltpu.async_copy(kb, p_out.at[pl.ds(bo, CH)], osem)
                pltpu.async_copy(vb, vs_out.at[pl.ds(bo, CH)], osem)
            return carry

        nt = NCHUNK // 3
        lax.fori_loop(0, nt, triple, 0)
        drain_outs()
        for t in range(NCHUNK - 3 * nt):
            bT = base0 + (3 * nt + t) * CH
            hA = issue(bT, siA, diA, kbA, qbA, vbA, gsA)
            for x in hA:
                x.wait()
            mul(kbA, qbA)
            pltpu.sync_copy(kbA, p_out.at[pl.ds(bT, CH)])
            pltpu.sync_copy(vbA, vs_out.at[pl.ds(bT, CH)])

    return body(k, q, v, src, dst)


def _scatter_stage(av, dst):
    @functools.partial(
        pl.kernel,
        mesh=_sc_mesh(),
        out_type=jax.ShapeDtypeStruct((NC, NPAD, 128), _f32),
        scratch_types=[pltpu.VMEM((CH,), jnp.int32),
                       pltpu.VMEM((CH, 128), _f32),
                       pltpu.VMEM((CH,), jnp.int32),
                       pltpu.VMEM((CH, 128), _f32),
                       pltpu.VMEM((128, 128), _f32),
                       pltpu.VMEM_SHARED((NPAD, 128), _f32),
                       pltpu.SemaphoreType.DMA,
                       pltpu.SemaphoreType.DMA,
                       pltpu.SemaphoreType.DMA],
    )
    def body(av_hbm, dst_hbm, wv_out, diA, avbA, diB, avbB, zb, wv_acc,
             rsA, rsB, ssem):
        c = lax.axis_index("c")
        s = lax.axis_index("s")
        wid = s * NC + c
        base0 = wid * EPT

        def zrow(r, carry):
            for j in range(8):
                zb[r, pl.ds(j * 16, 16)] = jnp.zeros((16,), _f32)
            return carry

        lax.fori_loop(0, 128, zrow, 0)
        for kk in range(NPT // 128):
            rows = pl.ds(s * NPT + kk * 128, 128)
            pltpu.sync_copy(zb, wv_acc.at[rows])
        plsc.subcore_barrier()

        def drain_adds():
            for _ in range(2):
                pltpu.make_async_copy(
                    av_hbm.at[pl.ds(base0, CH)], avbA, ssem).wait()

        def pair(i, carry):
            b0 = base0 + (2 * i) * CH
            b1 = b0 + CH

            @pl.when(i > 0)
            def _():
                drain_adds()

            pltpu.sync_copy(dst_hbm.at[pl.ds(b0, CH)], diA)
            hA = pltpu.async_copy(av_hbm.at[pl.ds(b0, CH)], avbA, rsA)
            pltpu.sync_copy(dst_hbm.at[pl.ds(b1, CH)], diB)
            hB = pltpu.async_copy(av_hbm.at[pl.ds(b1, CH)], avbB, rsB)
            hA.wait()
            pltpu.async_copy(avbA, wv_acc.at[diA], ssem, add=True)
            hB.wait()
            pltpu.async_copy(avbB, wv_acc.at[diB], ssem, add=True)
            return carry

        lax.fori_loop(0, NCHUNK // 2, pair, 0)
        drain_adds()
        bT = base0 + (NCHUNK - 1) * CH
        pltpu.sync_copy(dst_hbm.at[pl.ds(bT, CH)], diA)
        pltpu.sync_copy(av_hbm.at[pl.ds(bT, CH)], avbA)
        pltpu.sync_copy(avbA, wv_acc.at[diA], add=True)
        plsc.subcore_barrier()
        rows = pl.ds(s * NPT, NPT)
        pltpu.sync_copy(wv_acc.at[rows], wv_out.at[c, rows])

    return body(av, dst)


def _attn_stage(a16, zr128, dst):
    @functools.partial(
        pl.kernel,
        mesh=_sc_mesh(),
        out_type=jax.ShapeDtypeStruct((N_EDGES, 16), _f32),
        scratch_types=[pltpu.VMEM((CH,), jnp.int32),
                       pltpu.VMEM((CH, 16), _f32),
                       pltpu.VMEM((CH, 128), _f32),
                       pltpu.VMEM((CH,), jnp.int32),
                       pltpu.VMEM((CH, 16), _f32),
                       pltpu.VMEM((CH, 128), _f32),
                       pltpu.VMEM_SHARED((NPAD, 128), _f32),
                       pltpu.SemaphoreType.DMA,
                       pltpu.SemaphoreType.DMA,
                       pltpu.SemaphoreType.DMA],
    )
    def body(a_hbm, zr_hbm, dst_hbm, out_hbm, diA, abA, zrbA, diB, abB, zrbB,
             zr_sh, gsA, gsB, osem):
        c = lax.axis_index("c")
        s = lax.axis_index("s")
        wid = s * NC + c
        base0 = wid * EPT

        rows = pl.ds(s * NPT, NPT)
        pltpu.sync_copy(zr_hbm.at[rows], zr_sh.at[rows])
        plsc.subcore_barrier()

        def issue(base, di, ab, gsem):
            pltpu.sync_copy(dst_hbm.at[pl.ds(base, CH)], di)
            pltpu.sync_copy(a_hbm.at[pl.ds(base, CH)], ab)

        def mul(ab, zrb):
            def row(r, carry2):
                sl = pl.ds(0, 16)
                ab[r, sl] = ab[r, sl] * zrb[r, sl]
                return carry2

            lax.fori_loop(0, CH, row, 0)

        def pair(i, carry):
            b0 = base0 + (2 * i) * CH
            b1 = b0 + CH
            issue(b0, diA, abA, gsA)
            hA = pltpu.async_copy(zr_sh.at[diA], zrbA, gsA)
            issue(b1, diB, abB, gsB)
            hB = pltpu.async_copy(zr_sh.at[diB], zrbB, gsB)
            hA.wait()
            mul(abA, zrbA)
            o1 = pltpu.async_copy(abA, out_hbm.at[pl.ds(b0, CH)], osem)
            hB.wait()
            mul(abB, zrbB)
            o2 = pltpu.async_copy(abB, out_hbm.at[pl.ds(b1, CH)], osem)
            o1.wait()
            o2.wait()
            return carry

        lax.fori_loop(0, NCHUNK // 2, pair, 0)
        bT = base0 + (NCHUNK - 1) * CH
        issue(bT, diA, abA, gsA)
        pltpu.async_copy(zr_sh.at[diA], zrbA, gsA).wait()
        mul(abA, zrbA)
        pltpu.sync_copy(abA, out_hbm.at[pl.ds(bT, CH)])

    return body(a16, zr128, dst)


def kernel(atom, lap_pos_enc, bond, edge_index, params):
    src = edge_index[0].astype(jnp.int32)
    dst = edge_index[1].astype(jnp.int32)
    p = params

    def b2(x):
        return x.reshape(1, -1)

    h = _embed_h(atom, lap_pos_enc,
                 p['lin_h']['w'], b2(p['lin_h']['b']),
                 p['lap']['w'], b2(p['lap']['b']))
    e = _embed_e(bond, p['lin_e']['w'], b2(p['lin_e']['b']))

    attns = []
    for lp in p['layers']:
        q, k, v = _proj(h,
                        lp['Q']['w'], b2(lp['Q']['b']),
                        lp['K']['w'], b2(lp['K']['b']),
                        lp['V']['w'], b2(lp['V']['b']))
        P, Vs = _gather_stage(k, q, v, src, dst)
        e2, av, a16, a128 = _edge(e, P, Vs,
                            lp['E']['w'], b2(lp['E']['b']),
                            lp['Oe']['w'], b2(lp['Oe']['b']),
                            lp['ffn_e1']['w'], b2(lp['ffn_e1']['b']),
                            lp['ffn_e2']['w'], b2(lp['ffn_e2']['b']),
                            b2(lp['ln1e_g']), b2(lp['ln1e_b']),
                            b2(lp['ln2e_g']), b2(lp['ln2e_b']))
        wvp = _scatter_stage(av, dst)
        zp = _scatter_stage(a128, dst)
        h2, zr = _node(wvp, zp, h,
                       lp['O']['w'], b2(lp['O']['b']),
                       lp['ffn_h1']['w'], b2(lp['ffn_h1']['b']),
                       lp['ffn_h2']['w'], b2(lp['ffn_h2']['b']),
                       b2(lp['ln1h_g']), b2(lp['ln1h_b']),
                       b2(lp['ln2h_g']), b2(lp['ln2h_b']))
        at16 = _attn_stage(a16, zr, dst)
        attns.append(at16[:, :8])
        h, e = h2, e2

    return (h, e, tuple(attns))


# speedup vs baseline: 1.0135x; 1.0135x over previous
import functools

import jax
import jax.numpy as jnp
from jax import lax
from jax.experimental import pallas as pl
from jax.experimental.pallas import tpu as pltpu
from jax.experimental.pallas import tpu_sc as plsc

N_NODES = 10000
N_EDGES = 320000
HEADS = 8
DK = 16
D = 128

NC = 2
NS = 16
NW = NC * NS
EPT = N_EDGES // NW
CH = 128
NF = EPT // CH
CT = EPT - NF * CH
NPAD = 10240
NPT = NPAD // NS

_BN = 2000
_BE = 2560

_f32 = jnp.float32


def _ln(x, g, b):
    m = jnp.mean(x, axis=1, keepdims=True)
    v = jnp.mean((x - m) * (x - m), axis=1, keepdims=True)
    return (x - m) * lax.rsqrt(v + 1e-5) * g + b


def _dot(a, b):
    return jnp.dot(a, b, preferred_element_type=_f32)


def _bdot(a, b):
    return jnp.dot(a.astype(jnp.bfloat16), b.astype(jnp.bfloat16),
                   preferred_element_type=_f32)


def _embed_h_body(atom_ref, lap_ref, wh, bh, wl, bl, out_ref):
    out_ref[...] = (_dot(atom_ref[...], wh[...]) + bh[...]
                    + _dot(lap_ref[...], wl[...]) + bl[...])


def _embed_h(atom, lap, wh, bh, wl, bl):
    grid = (N_NODES // _BN,)
    return pl.pallas_call(
        _embed_h_body,
        grid=grid,
        in_specs=[
            pl.BlockSpec((_BN, 128), lambda i: (i, 0)),
            pl.BlockSpec((_BN, 16), lambda i: (i, 0)),
            pl.BlockSpec((128, 128), lambda i: (0, 0)),
            pl.BlockSpec((1, 128), lambda i: (0, 0)),
            pl.BlockSpec((16, 128), lambda i: (0, 0)),
            pl.BlockSpec((1, 128), lambda i: (0, 0)),
        ],
        out_specs=pl.BlockSpec((_BN, 128), lambda i: (i, 0)),
        out_shape=jax.ShapeDtypeStruct((N_NODES, 128), _f32),
    )(atom, lap, wh, bh, wl, bl)


def _embed_e_body(bond_ref, we, be, out_ref):
    out_ref[...] = _dot(bond_ref[...], we[...]) + be[...]


def _embed_e(bond, we, be):
    grid = (N_EDGES // _BE,)
    return pl.pallas_call(
        _embed_e_body,
        grid=grid,
        in_specs=[
            pl.BlockSpec((_BE, 16), lambda i: (i, 0)),
            pl.BlockSpec((16, 128), lambda i: (0, 0)),
            pl.BlockSpec((1, 128), lambda i: (0, 0)),
        ],
        out_specs=pl.BlockSpec((_BE, 128), lambda i: (i, 0)),
        out_shape=jax.ShapeDtypeStruct((N_EDGES, 128), _f32),
    )(bond, we, be)


def _proj_body(h_ref, wq, bq, wk, bk, wv, bv, q_ref, k_ref, v_ref):
    h = h_ref[...]
    q_ref[...] = (_dot(h, wq[...]) + bq[...]) * 0.25
    k_ref[...] = _dot(h, wk[...]) + bk[...]
    v_ref[...] = _dot(h, wv[...]) + bv[...]


def _proj(h, wq, bq, wk, bk, wv, bv):
    grid = (N_NODES // _BN,)
    wspec = pl.BlockSpec((128, 128), lambda i: (0, 0))
    bspec = pl.BlockSpec((1, 128), lambda i: (0, 0))
    ospec = pl.BlockSpec((_BN, 128), lambda i: (i, 0))
    oshape = jax.ShapeDtypeStruct((N_NODES, 128), _f32)
    return pl.pallas_call(
        _proj_body,
        grid=grid,
        in_specs=[pl.BlockSpec((_BN, 128), lambda i: (i, 0)),
                  wspec, bspec, wspec, bspec, wspec, bspec],
        out_specs=[ospec, ospec, ospec],
        out_shape=[oshape, oshape, oshape],
    )(h, wq, bq, wk, bk, wv, bv)


def _edge_body(e_ref, p_ref, vs_ref, we, be, woe, boe, w1, c1, w2, c2,
               g1, d1, g2, d2, e2_ref, av_ref, a_ref, a128_ref):
    e = e_ref[...]
    ee = _dot(e, we[...]) + be[...]
    score = p_ref[...] * ee
    parts = []
    for h in range(HEADS):
        parts.append(jnp.sum(score[:, h * DK:(h + 1) * DK], axis=1,
                             keepdims=True))
    s = jnp.concatenate(parts, axis=1)
    a = jnp.exp(jnp.clip(s, -5.0, 5.0))
    vs = vs_ref[...]
    av_parts = []
    for h in range(HEADS):
        av_parts.append(vs[:, h * DK:(h + 1) * DK] * a[:, h:h + 1])
    av_ref[...] = jnp.concatenate(av_parts, axis=1)
    a_ref[...] = jnp.concatenate([a, jnp.zeros_like(a)], axis=1)
    a128_ref[...] = jnp.concatenate(
        [a, jnp.zeros((a.shape[0], 120), _f32)], axis=1)
    e1 = _ln(e + _dot(score, woe[...]) + boe[...], g1[...], d1[...])
    ffn = _dot(jnp.maximum(_dot(e1, w1[...]) + c1[...], 0.0), w2[...]) + c2[...]
    e2_ref[...] = _ln(e1 + ffn, g2[...], d2[...])


def _edge(e, p, vs, we, be, woe, boe, w1, c1, w2, c2, g1, d1, g2, d2):
    grid = (N_EDGES // _BE,)
    espec = pl.BlockSpec((_BE, 128), lambda i: (i, 0))
    return pl.pallas_call(
        _edge_body,
        grid=grid,
        in_specs=[espec, espec, espec,
                  pl.BlockSpec((128, 128), lambda i: (0, 0)),
                  pl.BlockSpec((1, 128), lambda i: (0, 0)),
                  pl.BlockSpec((128, 128), lambda i: (0, 0)),
                  pl.BlockSpec((1, 128), lambda i: (0, 0)),
                  pl.BlockSpec((128, 256), lambda i: (0, 0)),
                  pl.BlockSpec((1, 256), lambda i: (0, 0)),
                  pl.BlockSpec((256, 128), lambda i: (0, 0)),
                  pl.BlockSpec((1, 128), lambda i: (0, 0)),
                  pl.BlockSpec((1, 128), lambda i: (0, 0)),
                  pl.BlockSpec((1, 128), lambda i: (0, 0)),
                  pl.BlockSpec((1, 128), lambda i: (0, 0)),
                  pl.BlockSpec((1, 128), lambda i: (0, 0))],
        out_specs=[espec, espec, pl.BlockSpec((_BE, 16), lambda i: (i, 0)),
                   espec],
        out_shape=[jax.ShapeDtypeStruct((N_EDGES, 128), _f32),
                   jax.ShapeDtypeStruct((N_EDGES, 128), _f32),
                   jax.ShapeDtypeStruct((N_EDGES, 16), _f32),
                   jax.ShapeDtypeStruct((N_EDGES, 128), _f32)],
    )(e, p, vs, we, be, woe, boe, w1, c1, w2, c2, g1, d1, g2, d2)


def _node_body(wvp_ref, zp_ref, h_ref, wo, bo, w1, c1, w2, c2,
               g1, d1, g2, d2, h2_ref, zr_ref):
    wv = wvp_ref[0] + wvp_ref[1]
    z = (zp_ref[0] + zp_ref[1])[:, :16]
    zr = 1.0 / (z + 1e-6)
    zr_ref[...] = jnp.concatenate(
        [zr, jnp.zeros((zr.shape[0], 112), _f32)], axis=1)
    parts = []
    for h in range(HEADS):
        parts.append(wv[:, h * DK:(h + 1) * DK] * zr[:, h:h + 1])
    h_attn = jnp.concatenate(parts, axis=1)
    hh = h_ref[...]
    h1 = _ln(hh + _dot(h_attn, wo[...]) + bo[...], g1[...], d1[...])
    ffn = _dot(jnp.maximum(_dot(h1, w1[...]) + c1[...], 0.0), w2[...]) + c2[...]
    h2_ref[...] = _ln(h1 + ffn, g2[...], d2[...])


def _node(wvp, zp, h, wo, bo, w1, c1, w2, c2, g1, d1, g2, d2):
    grid = (N_NODES // _BN,)
    return pl.pallas_call(
        _node_body,
        grid=grid,
        in_specs=[pl.BlockSpec((2, _BN, 128), lambda i: (0, i, 0)),
                  pl.BlockSpec((2, _BN, 128), lambda i: (0, i, 0)),
                  pl.BlockSpec((_BN, 128), lambda i: (i, 0)),
                  pl.BlockSpec((128, 128), lambda i: (0, 0)),
                  pl.BlockSpec((1, 128), lambda i: (0, 0)),
                  pl.BlockSpec((128, 256), lambda i: (0, 0)),
                  pl.BlockSpec((1, 256), lambda i: (0, 0)),
                  pl.BlockSpec((256, 128), lambda i: (0, 0)),
                  pl.BlockSpec((1, 128), lambda i: (0, 0)),
                  pl.BlockSpec((1, 128), lambda i: (0, 0)),
                  pl.BlockSpec((1, 128), lambda i: (0, 0)),
                  pl.BlockSpec((1, 128), lambda i: (0, 0)),
                  pl.BlockSpec((1, 128), lambda i: (0, 0))],
        out_specs=[pl.BlockSpec((_BN, 128), lambda i: (i, 0)),
                   pl.BlockSpec((_BN, 128), lambda i: (i, 0))],
        out_shape=[jax.ShapeDtypeStruct((N_NODES, 128), _f32),
                   jax.ShapeDtypeStruct((NPAD, 128), _f32)],
    )(wvp, zp, h, wo, bo, w1, c1, w2, c2, g1, d1, g2, d2)


def _sc_mesh():
    return plsc.VectorSubcoreMesh(core_axis_name="c", subcore_axis_name="s")


def _gather_stage(k, q, v, src, dst):
    @functools.partial(
        pl.kernel,
        mesh=_sc_mesh(),
        out_type=[jax.ShapeDtypeStruct((N_EDGES, 128), _f32),
                  jax.ShapeDtypeStruct((N_EDGES, 128), _f32)],
        scratch_types=[pltpu.VMEM((CH,), jnp.int32),
                       pltpu.VMEM((CH,), jnp.int32),
                       pltpu.VMEM((CH, 128), _f32),
                       pltpu.VMEM((CH, 128), _f32),
                       pltpu.VMEM((CH, 128), _f32),
                       pltpu.VMEM((CH,), jnp.int32),
                       pltpu.VMEM((CH,), jnp.int32),
                       pltpu.VMEM((CH, 128), _f32),
                       pltpu.VMEM((CH, 128), _f32),
                       pltpu.VMEM((CH, 128), _f32),
                       pltpu.VMEM((CT,), jnp.int32),
                       pltpu.VMEM((CT,), jnp.int32),
                       pltpu.VMEM((CT, 128), _f32),
                       pltpu.VMEM((CT, 128), _f32),
                       pltpu.VMEM((CT, 128), _f32),
                       pltpu.SemaphoreType.DMA,
                       pltpu.SemaphoreType.DMA,
                       pltpu.SemaphoreType.DMA],
    )
    def body(k_hbm, q_hbm, v_hbm, src_hbm, dst_hbm, p_out, vs_out,
             siA, diA, kbA, qbA, vbA, siB, diB, kbB, qbB, vbB,
             siT, diT, kbT, qbT, vbT, gsA, gsB, osem):
        wid = lax.axis_index("s") * NC + lax.axis_index("c")
        base0 = wid * EPT

        def issue(base, n, si, di, kb, qb, vb, gsem):
            pltpu.sync_copy(src_hbm.at[pl.ds(base, n)], si)
            pltpu.sync_copy(dst_hbm.at[pl.ds(base, n)], di)
            return (pltpu.async_copy(k_hbm.at[si], kb, gsem),
                    pltpu.async_copy(q_hbm.at[di], qb, gsem),
                    pltpu.async_copy(v_hbm.at[si], vb, gsem))

        def mul(n, kb, qb):
            def row(r, carry2):
                for j in range(8):
                    sl = pl.ds(j * 16, 16)
                    kb[r, sl] = kb[r, sl] * qb[r, sl]
                return carry2

            lax.fori_loop(0, n, row, 0)

        def drain_outs():
            for _ in range(4):
                pltpu.make_async_copy(
                    kbA, p_out.at[pl.ds(base0, CH)], osem).wait()

        def pair(i, carry):
            b0 = base0 + (2 * i) * CH

            @pl.when(i > 0)
            def _():
                drain_outs()

            hA = issue(b0, CH, siA, diA, kbA, qbA, vbA, gsA)
            hB = issue(b0 + CH, CH, siB, diB, kbB, qbB, vbB, gsB)
            for h, kb, qb, vb, bo in ((hA, kbA, qbA, vbA, b0),
                                      (hB, kbB, qbB, vbB, b0 + CH)):
                for x in h:
                    x.wait()
                mul(CH, kb, qb)
                pltpu.async_copy(kb, p_out.at[pl.ds(bo, CH)], osem)
                pltpu.async_copy(vb, vs_out.at[pl.ds(bo, CH)], osem)
            return carry

        lax.fori_loop(0, NF // 2, pair, 0)
        drain_outs()
        bT = base0 + NF * CH
        hT = issue(bT, CT, siT, diT, kbT, qbT, vbT, gsA)
        for x in hT:
            x.wait()
        mul(CT, kbT, qbT)
        pltpu.sync_copy(kbT, p_out.at[pl.ds(bT, CT)])
        pltpu.sync_copy(vbT, vs_out.at[pl.ds(bT, CT)])

    return body(k, q, v, src, dst)


def _scatter_stage(av, dst):
    @functools.partial(
        pl.kernel,
        mesh=_sc_mesh(),
        out_type=jax.ShapeDtypeStruct((NC, NPAD, 128), _f32),
        scratch_types=[pltpu.VMEM((CH,), jnp.int32),
                       pltpu.VMEM((CH, 128), _f32),
                       pltpu.VMEM((CH,), jnp.int32),
                       pltpu.VMEM((CH, 128), _f32),
                       pltpu.VMEM((CT,), jnp.int32),
                       pltpu.VMEM((CT, 128), _f32),
                       pltpu.VMEM((64, 128), _f32),
                       pltpu.VMEM_SHARED((NPAD, 128), _f32),
                       pltpu.SemaphoreType.DMA,
                       pltpu.SemaphoreType.DMA,
                       pltpu.SemaphoreType.DMA],
    )
    def body(av_hbm, dst_hbm, wv_out, diA, avbA, diB, avbB, diT, avbT,
             zb, wv_acc, rsA, rsB, ssem):
        c = lax.axis_index("c")
        s = lax.axis_index("s")
        wid = s * NC + c
        base0 = wid * EPT

        def zrow(r, carry):
            for j in range(8):
                zb[r, pl.ds(j * 16, 16)] = jnp.zeros((16,), _f32)
            return carry

        lax.fori_loop(0, 64, zrow, 0)
        for kk in range(NPT // 64):
            rows = pl.ds(s * NPT + kk * 64, 64)
            pltpu.sync_copy(zb, wv_acc.at[rows])
        plsc.subcore_barrier()

        def drain_adds():
            for _ in range(2):
                pltpu.make_async_copy(
                    av_hbm.at[pl.ds(base0, CH)], avbA, ssem).wait()

        def pair(i, carry):
            b0 = base0 + (2 * i) * CH
            b1 = b0 + CH

            @pl.when(i > 0)
            def _():
                drain_adds()

            pltpu.sync_copy(dst_hbm.at[pl.ds(b0, CH)], diA)
            hA = pltpu.async_copy(av_hbm.at[pl.ds(b0, CH)], avbA, rsA)
            pltpu.sync_copy(dst_hbm.at[pl.ds(b1, CH)], diB)
            hB = pltpu.async_copy(av_hbm.at[pl.ds(b1, CH)], avbB, rsB)
            hA.wait()
            pltpu.async_copy(avbA, wv_acc.at[diA], ssem, add=True)
            hB.wait()
            pltpu.async_copy(avbB, wv_acc.at[diB], ssem, add=True)
            return carry

        lax.fori_loop(0, NF // 2, pair, 0)
        drain_adds()
        bT = base0 + NF * CH
        pltpu.sync_copy(dst_hbm.at[pl.ds(bT, CT)], diT)
        pltpu.sync_copy(av_hbm.at[pl.ds(bT, CT)], avbT)
        pltpu.sync_copy(avbT, wv_acc.at[diT], add=True)
        plsc.subcore_barrier()
        rows = pl.ds(s * NPT, NPT)
        pltpu.sync_copy(wv_acc.at[rows], wv_out.at[c, rows])

    return body(av, dst)


def _attn_stage(a16, zr128, dst):
    @functools.partial(
        pl.kernel,
        mesh=_sc_mesh(),
        out_type=jax.ShapeDtypeStruct((N_EDGES, 16), _f32),
        scratch_types=[pltpu.VMEM((CH,), jnp.int32),
                       pltpu.VMEM((CH, 16), _f32),
                       pltpu.VMEM((CH, 128), _f32),
                       pltpu.VMEM((CH,), jnp.int32),
                       pltpu.VMEM((CH, 16), _f32),
                       pltpu.VMEM((CH, 128), _f32),
                       pltpu.VMEM((CT,), jnp.int32),
                       pltpu.VMEM((CT, 16), _f32),
                       pltpu.VMEM((CT, 128), _f32),
                       pltpu.SemaphoreType.DMA,
                       pltpu.SemaphoreType.DMA,
                       pltpu.SemaphoreType.DMA],
    )
    def body(a_hbm, zr_hbm, dst_hbm, out_hbm, diA, abA, zrbA, diB, abB, zrbB,
             diT, abT, zrbT, gsA, gsB, osem):
        wid = lax.axis_index("s") * NC + lax.axis_index("c")
        base0 = wid * EPT

        def issue(base, n, di, ab, gsem):
            pltpu.sync_copy(dst_hbm.at[pl.ds(base, n)], di)
            pltpu.sync_copy(a_hbm.at[pl.ds(base, n)], ab)

        def mul(n, ab, zrb):
            def row(r, carry2):
                sl = pl.ds(0, 16)
                ab[r, sl] = ab[r, sl] * zrb[r, sl]
                return carry2

            lax.fori_loop(0, n, row, 0)

        def drain_outs():
            for _ in range(2):
                pltpu.make_async_copy(
                    abA, out_hbm.at[pl.ds(base0, CH)], osem).wait()

        def pair(i, carry):
            b0 = base0 + (2 * i) * CH
            b1 = b0 + CH

            @pl.when(i > 0)
            def _():
                drain_outs()

            issue(b0, CH, diA, abA, gsA)
            hA = pltpu.async_copy(zr_hbm.at[diA], zrbA, gsA)
            issue(b1, CH, diB, abB, gsB)
            hB = pltpu.async_copy(zr_hbm.at[diB], zrbB, gsB)
            hA.wait()
            mul(CH, abA, zrbA)
            pltpu.async_copy(abA, out_hbm.at[pl.ds(b0, CH)], osem)
            hB.wait()
            mul(CH, abB, zrbB)
            pltpu.async_copy(abB, out_hbm.at[pl.ds(b1, CH)], osem)
            return carry

        lax.fori_loop(0, NF // 2, pair, 0)
        drain_outs()
        bT = base0 + NF * CH
        issue(bT, CT, diT, abT, gsA)
        pltpu.async_copy(zr_hbm.at[diT], zrbT, gsA).wait()
        mul(CT, abT, zrbT)
        pltpu.sync_copy(abT, out_hbm.at[pl.ds(bT, CT)])

    return body(a16, zr128, dst)


def kernel(atom, lap_pos_enc, bond, edge_index, params):
    src = edge_index[0].astype(jnp.int32)
    dst = edge_index[1].astype(jnp.int32)
    p = params

    def b2(x):
        return x.reshape(1, -1)

    h = _embed_h(atom, lap_pos_enc,
                 p['lin_h']['w'], b2(p['lin_h']['b']),
                 p['lap']['w'], b2(p['lap']['b']))
    e = _embed_e(bond, p['lin_e']['w'], b2(p['lin_e']['b']))

    attns = []
    for lp in p['layers']:
        q, k, v = _proj(h,
                        lp['Q']['w'], b2(lp['Q']['b']),
                        lp['K']['w'], b2(lp['K']['b']),
                        lp['V']['w'], b2(lp['V']['b']))
        P, Vs = _gather_stage(k, q, v, src, dst)
        e2, av, a16, a128 = _edge(e, P, Vs,
                            lp['E']['w'], b2(lp['E']['b']),
                            lp['Oe']['w'], b2(lp['Oe']['b']),
                            lp['ffn_e1']['w'], b2(lp['ffn_e1']['b']),
                            lp['ffn_e2']['w'], b2(lp['ffn_e2']['b']),
                            b2(lp['ln1e_g']), b2(lp['ln1e_b']),
                            b2(lp['ln2e_g']), b2(lp['ln2e_b']))
        wvp = _scatter_stage(av, dst)
        zp = _scatter_stage(a128, dst)
        h2, zr = _node(wvp, zp, h,
                       lp['O']['w'], b2(lp['O']['b']),
                       lp['ffn_h1']['w'], b2(lp['ffn_h1']['b']),
                       lp['ffn_h2']['w'], b2(lp['ffn_h2']['b']),
                       b2(lp['ln1h_g']), b2(lp['ln1h_b']),
                       b2(lp['ln2h_g']), b2(lp['ln2h_b']))
        at16 = _attn_stage(a16, zr, dst)
        attns.append(at16[:, :8])
        h, e = h2, e2

    return (h, e, tuple(attns))
